# baseline (speedup 1.0000x reference)
import functools

import jax
import jax.numpy as jnp
from jax import lax
from jax.experimental import pallas as pl
from jax.experimental.pallas import tpu as pltpu
from jax.experimental.pallas import tpu_sc as plsc

_IDX_BLK = 128


def kernel(user, item, user_emb, item_emb, user_bias, item_bias, global_bias):
    B = user.shape[0]
    D = user_emb.shape[1]
    info = plsc.get_sparse_core_info()
    nc, ns, L = info.num_cores, info.num_subcores, info.num_lanes
    nw = nc * ns
    bpw = B // nw
    nblk = bpw // _IDX_BLK
    nchunk = bpw // L

    user2d = user.reshape(B // _IDX_BLK, _IDX_BLK).astype(jnp.int32)
    item2d = item.reshape(B // _IDX_BLK, _IDX_BLK).astype(jnp.int32)
    ubf = user_bias.reshape(-1)
    ibf = item_bias.reshape(-1)
    gb16 = jnp.broadcast_to(global_bias.astype(jnp.float32), (L,))

    mesh = plsc.VectorSubcoreMesh(core_axis_name="c", subcore_axis_name="s")

    @functools.partial(
        pl.kernel,
        mesh=mesh,
        out_type=jax.ShapeDtypeStruct((B,), jnp.float32),
        compiler_params=pltpu.CompilerParams(
            needs_layout_passes=False, use_tc_tiling_on_sc=False),
        scratch_types=[
            pltpu.VMEM((nblk, _IDX_BLK), jnp.int32),
            pltpu.VMEM((nblk, _IDX_BLK), jnp.int32),
            pltpu.VMEM((bpw, D), jnp.float32),
            pltpu.VMEM((bpw, D), jnp.float32),
            pltpu.VMEM((bpw,), jnp.float32),
            pltpu.VMEM((bpw,), jnp.float32),
            pltpu.VMEM((bpw,), jnp.float32),
            pltpu.VMEM((L,), jnp.float32),
            pltpu.SemaphoreType.DMA,
        ],
    )
    def mf(user_hbm, item_hbm, ue_hbm, ie_hbm, ub_hbm, ib_hbm, gb_hbm, out_hbm,
           uidx_v, iidx_v, ue_v, ie_v, ub_v, ib_v, out_v, gb_v, sem):
        wid = lax.axis_index("s") * nc + lax.axis_index("c")
        row0 = wid * nblk
        base = wid * bpw
        pltpu.sync_copy(user_hbm.at[pl.ds(row0, nblk), :], uidx_v)
        pltpu.sync_copy(item_hbm.at[pl.ds(row0, nblk), :], iidx_v)
        pltpu.sync_copy(gb_hbm, gb_v)
        copies = []
        for j in range(nblk):
            sl = pl.ds(j * _IDX_BLK, _IDX_BLK)
            copies.append(
                pltpu.async_copy(ue_hbm.at[uidx_v.at[j]], ue_v.at[sl, :], sem))
            copies.append(
                pltpu.async_copy(ie_hbm.at[iidx_v.at[j]], ie_v.at[sl, :], sem))
            copies.append(
                pltpu.async_copy(ub_hbm.at[uidx_v.at[j]], ub_v.at[sl], sem))
            copies.append(
                pltpu.async_copy(ib_hbm.at[iidx_v.at[j]], ib_v.at[sl], sem))
        for cp in copies:
            cp.wait()
        gvec = gb_v[...]

        def chunk(c, carry):
            r0 = c * L
            rows = r0 + lax.iota(jnp.int32, L)
            accs = [ub_v[pl.ds(r0, L)] + ib_v[pl.ds(r0, L)] + gvec,
                    jnp.zeros((L,), jnp.float32),
                    jnp.zeros((L,), jnp.float32),
                    jnp.zeros((L,), jnp.float32)]
            for d in range(D):
                cols = jnp.full((L,), d, jnp.int32)
                u = plsc.load_gather(ue_v, [rows, cols])
                v = plsc.load_gather(ie_v, [rows, cols])
                accs[d % 4] = accs[d % 4] + u * v
            s = (accs[0] + accs[1]) + (accs[2] + accs[3])
            out_v[pl.ds(r0, L)] = 1.0 / (1.0 + jnp.exp(-s))
            return carry

        lax.fori_loop(0, nchunk, chunk, 0)
        pltpu.sync_copy(out_v, out_hbm.at[pl.ds(base, bpw)])

    return mf(user2d, item2d, user_emb, item_emb, ubf, ibf, gb16)
